# Initial kernel scaffold; baseline (speedup 1.0000x reference)
import jax
import jax.numpy as jnp
from jax.experimental import pallas as pl


def kernel(fx, dx, pair_i, pair_j):
    raise NotImplementedError("write your pallas kernel here")


# speedup vs baseline: 19.6222x; 19.6222x over previous
import jax
import jax.numpy as jnp
from jax import lax
from jax.experimental import pallas as pl
from jax.experimental.pallas import tpu as pltpu
from jax.experimental.pallas import tpu_sc as plsc

N_ATOMS = 100000
N_PAD = 100096
ATOM_SLICE = N_PAD // 16
N_EDGES = 6400000

NC = 2
NS = 16
NW = NC * NS

RPS = 128
K = 8
CHUNK = K * RPS
TOTAL_CHUNKS = N_EDGES // CHUNK
BASE_CHUNKS = TOTAL_CHUNKS // NW
EXTRA = TOTAL_CHUNKS - BASE_CHUNKS * NW


def _sc_body(dxx_hbm, dxy_hbm, dxz_hbm, ii_hbm, jj_hbm, zeros_hbm,
             oxi_hbm, oyi_hbm, ozi_hbm, oxj_hbm, oyj_hbm, ozj_hbm,
             axi, ayi, azi, axj, ayj, azj,
             idx_i_v, idx_j_v, vx, vy, vz):
    cid = lax.axis_index("c")
    sid = lax.axis_index("s")
    wid = sid * NC + cid

    a0 = pl.multiple_of(sid * ATOM_SLICE, 8)
    for acc in (axi, ayi, azi, axj, ayj, azj):
        pltpu.sync_copy(zeros_hbm, acc.at[pl.ds(a0, ATOM_SLICE)])
    plsc.subcore_barrier()

    chunk0 = wid * BASE_CHUNKS + lax.min(wid, EXTRA)
    n_chunks = BASE_CHUNKS + jnp.where(wid < EXTRA, 1, 0)

    def chunk_body(g, carry):
        c = chunk0 + g
        base_r = pl.multiple_of(c * K, 8)
        base_e = pl.multiple_of(c * CHUNK, 8)
        pltpu.sync_copy(ii_hbm.at[pl.ds(base_r, K)], idx_i_v)
        pltpu.sync_copy(jj_hbm.at[pl.ds(base_r, K)], idx_j_v)
        pltpu.sync_copy(dxx_hbm.at[pl.ds(base_e, CHUNK)], vx)
        pltpu.sync_copy(dxy_hbm.at[pl.ds(base_e, CHUNK)], vy)
        pltpu.sync_copy(dxz_hbm.at[pl.ds(base_e, CHUNK)], vz)
        for k in range(K):
            s = pl.ds(k * RPS, RPS)
            ri = idx_i_v.at[k]
            rj = idx_j_v.at[k]
            pltpu.sync_copy(vx.at[s], axi.at[ri], add=True)
            pltpu.sync_copy(vy.at[s], ayi.at[ri], add=True)
            pltpu.sync_copy(vz.at[s], azi.at[ri], add=True)
            pltpu.sync_copy(vx.at[s], axj.at[rj], add=True)
            pltpu.sync_copy(vy.at[s], ayj.at[rj], add=True)
            pltpu.sync_copy(vz.at[s], azj.at[rj], add=True)
        return carry

    lax.fori_loop(0, n_chunks, chunk_body, 0)
    plsc.subcore_barrier()

    sl = pl.ds(a0, ATOM_SLICE)
    pltpu.sync_copy(axi.at[sl], oxi_hbm.at[cid].at[sl])
    pltpu.sync_copy(ayi.at[sl], oyi_hbm.at[cid].at[sl])
    pltpu.sync_copy(azi.at[sl], ozi_hbm.at[cid].at[sl])
    pltpu.sync_copy(axj.at[sl], oxj_hbm.at[cid].at[sl])
    pltpu.sync_copy(ayj.at[sl], oyj_hbm.at[cid].at[sl])
    pltpu.sync_copy(azj.at[sl], ozj_hbm.at[cid].at[sl])


_part = jax.ShapeDtypeStruct((NC, N_PAD), jnp.float32)
_sc_scatter = pl.kernel(
    _sc_body,
    out_type=(_part,) * 6,
    mesh=plsc.VectorSubcoreMesh(core_axis_name="c", subcore_axis_name="s",
                                num_cores=NC, num_subcores=NS),
    compiler_params=pltpu.CompilerParams(use_tc_tiling_on_sc=False),
    scratch_types=[
        pltpu.VMEM_SHARED((N_PAD,), jnp.float32),
        pltpu.VMEM_SHARED((N_PAD,), jnp.float32),
        pltpu.VMEM_SHARED((N_PAD,), jnp.float32),
        pltpu.VMEM_SHARED((N_PAD,), jnp.float32),
        pltpu.VMEM_SHARED((N_PAD,), jnp.float32),
        pltpu.VMEM_SHARED((N_PAD,), jnp.float32),
        pltpu.VMEM((K, RPS), jnp.int32),
        pltpu.VMEM((K, RPS), jnp.int32),
        pltpu.VMEM((CHUNK,), jnp.float32),
        pltpu.VMEM((CHUNK,), jnp.float32),
        pltpu.VMEM((CHUNK,), jnp.float32),
    ],
)


def _tc_combine(xi_ref, yi_ref, zi_ref, xj_ref, yj_ref, zj_ref, o_ref):
    o_ref[0, :] = (xi_ref[0] + xi_ref[1]) - (xj_ref[0] + xj_ref[1])
    o_ref[1, :] = (yi_ref[0] + yi_ref[1]) - (yj_ref[0] + yj_ref[1])
    o_ref[2, :] = (zi_ref[0] + zi_ref[1]) - (zj_ref[0] + zj_ref[1])


def kernel(fx, dx, pair_i, pair_j):
    del fx
    ii = pair_i.astype(jnp.int32).reshape(N_EDGES // RPS, RPS)
    jj = pair_j.astype(jnp.int32).reshape(N_EDGES // RPS, RPS)
    dxx = dx[:, 0]
    dxy = dx[:, 1]
    dxz = dx[:, 2]
    zeros = jnp.zeros((ATOM_SLICE,), jnp.float32)
    parts = _sc_scatter(dxx, dxy, dxz, ii, jj, zeros)
    out = pl.pallas_call(
        _tc_combine,
        out_shape=jax.ShapeDtypeStruct((3, N_PAD), jnp.float32),
    )(*parts)
    return out.T[:N_ATOMS]
